# Initial kernel scaffold; baseline (speedup 1.0000x reference)
import jax
import jax.numpy as jnp
from jax.experimental import pallas as pl


def kernel(pc, params):
    raise NotImplementedError("write your pallas kernel here")


# speedup vs baseline: 14.3643x; 14.3643x over previous
import functools
from functools import partial

import jax
import jax.numpy as jnp
from jax import lax
from jax.experimental import pallas as pl
from jax.experimental.pallas import tpu as pltpu
from jax.experimental.pallas import tpu_sc as plsc


def _fps_body(S, x_ref, nxr_ref):
    B, _, N = x_ref.shape
    xr = x_ref[:, 0, :]
    yr = x_ref[:, 1, :]
    zr = x_ref[:, 2, :]
    iota = lax.broadcasted_iota(jnp.int32, (B, N), 1)

    def step(s, carry):
        dist, far = carry
        oh = iota == far
        cx = jnp.sum(jnp.where(oh, xr, 0.0), axis=1, keepdims=True)
        cy = jnp.sum(jnp.where(oh, yr, 0.0), axis=1, keepdims=True)
        cz = jnp.sum(jnp.where(oh, zr, 0.0), axis=1, keepdims=True)
        nxr_ref[:, pl.ds(s, 1), :] = jnp.concatenate(
            [cx, cy, cz], axis=1)[:, None, :]
        dx = xr - cx
        dy = yr - cy
        dz = zr - cz
        d = (dx * dx + dy * dy) + dz * dz
        dist = jnp.minimum(dist, d)
        m = jnp.max(dist, axis=1, keepdims=True)
        far = jnp.min(jnp.where(dist == m, iota, N), axis=1, keepdims=True)
        return dist, far

    dist0 = jnp.full((B, N), 1e10, dtype=jnp.float32)
    far0 = jnp.zeros((B, 1), dtype=jnp.int32)
    lax.fori_loop(0, S, step, (dist0, far0))


def _fps(xyz_cm, S):
    B, _, N = xyz_cm.shape
    return pl.pallas_call(
        partial(_fps_body, S),
        out_shape=jax.ShapeDtypeStruct((B, S, 3), jnp.float32),
    )(xyz_cm)


def _cumsum_lanes(x):
    R, N = x.shape
    lane = lax.broadcasted_iota(jnp.int32, (R, N), 1)
    off = 1
    while off < N:
        sh = pltpu.roll(x, off, 1)
        x = x + jnp.where(lane >= off, sh, 0)
        off *= 2
    return x


def _bq_body(r2, K, N, x_ref, nxr_ref, out_ref):
    b = pl.program_id(0)
    BS = nxr_ref.shape[0]
    d2 = None
    for c in range(3):
        xc = x_ref[pl.ds(c, 1), :]
        nc = nxr_ref[:, pl.ds(c, 1)]
        df = nc - xc
        d2 = df * df if d2 is None else d2 + df * df
    mask = d2 <= r2
    cs = _cumsum_lanes(mask.astype(jnp.int32))
    a = jnp.where(mask, cs, 0)
    cnt = cs[:, N - 1:N]
    iota = lax.broadcasted_iota(jnp.int32, (BS, N), 1)
    base = b * N

    cols = []
    first = None
    for k in range(K):
        sel = a == (k + 1)
        idxk = jnp.sum(jnp.where(sel, iota, 0), axis=1, keepdims=True)
        if k == 0:
            first = idxk
        else:
            idxk = jnp.where(k < cnt, idxk, first)
        cols.append(idxk)
    out_ref[...] = jnp.concatenate(cols, axis=1) + base


def _ballquery(xyz_cm, nx_rm, radius, K):
    B, _, N = xyz_cm.shape
    S = nx_rm.shape[1]
    BS = min(S, 256)
    r2 = float(radius ** 2)
    return pl.pallas_call(
        partial(_bq_body, r2, K, N),
        grid=(B, S // BS),
        in_specs=[
            pl.BlockSpec((None, 3, N), lambda b, t: (b, 0, 0)),
            pl.BlockSpec((None, BS, 3), lambda b, t: (b, t, 0)),
        ],
        out_specs=pl.BlockSpec((None, BS, K), lambda b, t: (b, t, 0)),
        out_shape=jax.ShapeDtypeStruct((B, S, K), jnp.int32),
    )(xyz_cm, nx_rm)


def _sc_gather(table, idx_flat, Cp):
    M = idx_flat.shape[0]
    NW = 32
    CW = min(128, M // NW)
    CH = M // (NW * CW)
    idx3 = idx_flat.reshape(NW, CH, CW)
    mesh = plsc.VectorSubcoreMesh(core_axis_name="c", subcore_axis_name="s")

    @functools.partial(
        pl.kernel,
        mesh=mesh,
        compiler_params=pltpu.CompilerParams(use_tc_tiling_on_sc=False),
        out_type=jax.ShapeDtypeStruct((NW * CH, CW, Cp), jnp.float32),
        scratch_types=[
            pltpu.VMEM((CH, CW), jnp.int32),
            pltpu.VMEM((CW, Cp), jnp.float32),
            pltpu.SemaphoreType.DMA,
        ],
    )
    def k(table_hbm, idx_hbm, out_hbm, idx_v, rows_v, sem):
        wid = lax.axis_index("s") * 2 + lax.axis_index("c")
        pltpu.sync_copy(idx_hbm.at[wid], idx_v)

        def chunk(i, _):
            pltpu.async_copy(table_hbm.at[idx_v.at[i]], rows_v, sem).wait()
            pltpu.sync_copy(rows_v, out_hbm.at[wid * CH + i])
            return 0

        lax.fori_loop(0, CH, chunk, 0)

    out = k(table, idx3)
    return out.reshape(M, Cp)


def _affine(st_ref, g_ref, be_ref, cnt):
    mean = st_ref[0:1, :] / cnt
    var = st_ref[1:2, :] / cnt - mean * mean
    sc = g_ref[...] * lax.rsqrt(var + 1e-5)
    t = be_ref[...] - sc * mean
    return sc, t


def _acc_stats(st_ref, x):
    first = (pl.program_id(0) == 0) & (pl.program_id(1) == 0)

    @pl.when(first)
    def _():
        st_ref[...] = jnp.zeros_like(st_ref)

    s = jnp.sum(x, axis=0, keepdims=True)
    q = jnp.sum(x * x, axis=0, keepdims=True)
    st_ref[...] += jnp.concatenate([s, q], axis=0)


def _mlp1_body(extra, D, raw_ref, nx_ref, w_ref, x1_ref, st_ref):
    TS, K, Cp = raw_ref.shape
    gx = raw_ref[:, :, 0:3]
    diff = gx - nx_ref[...][:, None, :]
    if extra == 3:
        ex = diff
    else:
        d2 = jnp.sum(diff * diff, axis=-1, keepdims=True)
        ex = jnp.sqrt(d2 + 1e-6)
    pts = raw_ref[:, :, 3:3 + D]
    feats = jnp.concatenate([ex, pts], axis=-1)
    C0 = extra + D
    h = feats.reshape(TS * K, C0)
    x1 = jax.lax.dot_general(h, w_ref[...], (((1,), (0,)), ((), ())),
                             preferred_element_type=jnp.float32)
    x1_ref[...] = x1.reshape(TS, K, -1)
    _acc_stats(st_ref, x1)


def _mlp_mid_body(cnt, x_ref, st_in_ref, g_ref, be_ref, w_ref,
                  x2_ref, st_ref):
    TS, K, Cin = x_ref.shape
    sc, t = _affine(st_in_ref, g_ref, be_ref, cnt)
    h = jnp.maximum(x_ref[...].reshape(TS * K, Cin) * sc + t, 0.0)
    x2 = jax.lax.dot_general(h, w_ref[...], (((1,), (0,)), ((), ())),
                             preferred_element_type=jnp.float32)
    x2_ref[...] = x2.reshape(TS, K, -1)
    _acc_stats(st_ref, x2)


def _mlp_last_body(cnt, x_ref, st_in_ref, g_ref, be_ref, w_ref,
                   mx_ref, mn_ref, st_ref):
    TS, K, Cin = x_ref.shape
    sc, t = _affine(st_in_ref, g_ref, be_ref, cnt)
    h = jnp.maximum(x_ref[...].reshape(TS * K, Cin) * sc + t, 0.0)
    x3 = jax.lax.dot_general(h, w_ref[...], (((1,), (0,)), ((), ())),
                             preferred_element_type=jnp.float32)
    x3r = x3.reshape(TS, K, -1)
    mx_ref[...] = jnp.max(x3r, axis=1)
    mn_ref[...] = jnp.min(x3r, axis=1)
    _acc_stats(st_ref, x3)


def _finalize_body(cnt, mx_ref, mn_ref, st_ref, g_ref, be_ref, out_ref):
    sc, t = _affine(st_ref, g_ref, be_ref, cnt)
    v = jnp.where(sc >= 0.0, mx_ref[...], mn_ref[...])
    out_ref[...] = jnp.maximum(v * sc[None, :, :] + t[None, :, :], 0.0)


def _mlp(raw, nx_rm, lp, extra, D, TS):
    B, S, K, Cp = raw.shape
    (w1, _, g1, be1), (w2, _, g2, be2), (w3, _, g3, be3) = lp
    C1, C2, C3 = w1.shape[0], w2.shape[0], w3.shape[0]
    cnt = float(B * S * K)
    grid = (B, S // TS)
    stspec = pl.BlockSpec((2, C1), lambda b, t: (0, 0))
    full = lambda C: pl.BlockSpec((1, C), lambda b, t: (0, 0))

    x1, st1 = pl.pallas_call(
        partial(_mlp1_body, extra, D),
        grid=grid,
        in_specs=[
            pl.BlockSpec((None, TS, K, Cp), lambda b, t: (b, t, 0, 0)),
            pl.BlockSpec((None, TS, 3), lambda b, t: (b, t, 0)),
            pl.BlockSpec(w1.shape[::-1], lambda b, t: (0, 0)),
        ],
        out_specs=[
            pl.BlockSpec((None, TS, K, C1), lambda b, t: (b, t, 0, 0)),
            pl.BlockSpec((2, C1), lambda b, t: (0, 0)),
        ],
        out_shape=[
            jax.ShapeDtypeStruct((B, S, K, C1), jnp.float32),
            jax.ShapeDtypeStruct((2, C1), jnp.float32),
        ],
    )(raw, nx_rm, w1.T)

    x2, st2 = pl.pallas_call(
        partial(_mlp_mid_body, cnt),
        grid=grid,
        in_specs=[
            pl.BlockSpec((None, TS, K, C1), lambda b, t: (b, t, 0, 0)),
            pl.BlockSpec((2, C1), lambda b, t: (0, 0)),
            full(C1), full(C1),
            pl.BlockSpec(w2.shape[::-1], lambda b, t: (0, 0)),
        ],
        out_specs=[
            pl.BlockSpec((None, TS, K, C2), lambda b, t: (b, t, 0, 0)),
            pl.BlockSpec((2, C2), lambda b, t: (0, 0)),
        ],
        out_shape=[
            jax.ShapeDtypeStruct((B, S, K, C2), jnp.float32),
            jax.ShapeDtypeStruct((2, C2), jnp.float32),
        ],
    )(x1, st1, g1.reshape(1, -1), be1.reshape(1, -1), w2.T)

    mx, mn, st3 = pl.pallas_call(
        partial(_mlp_last_body, cnt),
        grid=grid,
        in_specs=[
            pl.BlockSpec((None, TS, K, C2), lambda b, t: (b, t, 0, 0)),
            pl.BlockSpec((2, C2), lambda b, t: (0, 0)),
            full(C2), full(C2),
            pl.BlockSpec(w3.shape[::-1], lambda b, t: (0, 0)),
        ],
        out_specs=[
            pl.BlockSpec((None, TS, C3), lambda b, t: (b, t, 0)),
            pl.BlockSpec((None, TS, C3), lambda b, t: (b, t, 0)),
            pl.BlockSpec((2, C3), lambda b, t: (0, 0)),
        ],
        out_shape=[
            jax.ShapeDtypeStruct((B, S, C3), jnp.float32),
            jax.ShapeDtypeStruct((B, S, C3), jnp.float32),
            jax.ShapeDtypeStruct((2, C3), jnp.float32),
        ],
    )(x2, st2, g2.reshape(1, -1), be2.reshape(1, -1), w3.T)

    return pl.pallas_call(
        partial(_finalize_body, cnt),
        out_shape=jax.ShapeDtypeStruct((B, S, C3), jnp.float32),
    )(mx, mn, st3, g3.reshape(1, -1), be3.reshape(1, -1))


_CFGS = [(1024, 0.1, 32, 3, 128), (256, 0.2, 32, 1, 64),
         (64, 0.4, 32, 1, 32), (16, 0.8, 32, 1, 16)]


def kernel(pc, params):
    B = pc.shape[0]
    xyz_cm = pc[:, :3, :]
    pts_rm = jnp.transpose(pc, (0, 2, 1))
    xyz_rm = pts_rm[:, :, :3]
    for (S, radius, K, extra, TS), lp in zip(_CFGS, params):
        N = xyz_cm.shape[2]
        D = pts_rm.shape[2]
        nx_rm = _fps(xyz_cm, S)
        nx_cm = jnp.transpose(nx_rm, (0, 2, 1))
        gidx = _ballquery(xyz_cm, nx_rm, radius, K)
        C = 3 + D
        Cp = -(-C // 16) * 16
        table = jnp.concatenate([xyz_rm, pts_rm], axis=-1)
        table = jnp.pad(table, ((0, 0), (0, 0), (0, Cp - C)))
        raw = _sc_gather(table.reshape(B * N, Cp), gidx.reshape(-1), Cp)
        raw = raw.reshape(B, S, K, Cp)
        pts_rm = _mlp(raw, nx_rm, lp, extra, D, TS)
        xyz_rm, xyz_cm = nx_rm, nx_cm
    return jnp.transpose(pts_rm, (0, 2, 1)).reshape(-1, 512)
